# baseline (speedup 1.0000x reference)
import jax
import jax.numpy as jnp
from jax import lax
from jax.experimental import pallas as pl
from jax.experimental.pallas import tpu as pltpu
from jax.experimental.pallas import tpu_sc as plsc

NUM_EXPERTS = 64
K = 8
BS = 32768
NUM_CORES = 2
NUM_SUBCORES = 16
NW = NUM_CORES * NUM_SUBCORES
BPW = BS // NW
IDX_CHUNK = 128
NCHUNK = BPW // IDX_CHUNK
HCHUNK = NCHUNK // 2
LANES = 16
ETILES = NUM_EXPERTS // 8


def _body(ids_hbm, w0_hbm, w1_hbm, out_hbm, ids_v, b0_v, b1_v, out_v,
          sem_a, sem_b, osem):
    c = lax.axis_index("c")
    s = lax.axis_index("s")
    wid = c * NUM_SUBCORES + s

    pltpu.sync_copy(ids_hbm.at[wid], ids_v)

    gathers = {0: [], 1: []}
    for j in range(NCHUNK):
        hf = j // HCHUNK
        sem = sem_a if hf == 0 else sem_b
        sl = pl.ds(j * IDX_CHUNK, IDX_CHUNK)
        gathers[hf].append(
            pltpu.async_copy(w0_hbm.at[ids_v.at[j]], b0_v.at[sl], sem)
        )
        gathers[hf].append(
            pltpu.async_copy(w1_hbm.at[ids_v.at[j]], b1_v.at[sl], sem)
        )

    zeros = jnp.zeros((LANES,), jnp.int32)

    def zero_body(r, carry):
        for cc in range(8):
            for k in range(8):
                out_v[r, cc, pl.ds(k * LANES, LANES)] = zeros
        return carry

    lane = lax.broadcasted_iota(jnp.int32, (LANES,), 0)
    ones = jnp.full((LANES,), 1, jnp.int32)
    m56 = jnp.full((LANES,), 56, jnp.int32)
    m7 = jnp.full((LANES,), 7, jnp.int32)

    def scat_body(i, carry):
        tc = lax.shift_right_logical(i, 3)
        t128 = lane + lax.shift_left(i & 7, 4)
        sl = pl.ds(LANES * i, LANES)
        for bv in (b0_v, b1_v):
            v = bv[sl]
            for m in range(4):
                vs = lax.shift_right_logical(v, 8 * m) if m else v
                plsc.store_scatter(
                    out_v, [(vs & m56) + tc, vs & m7, t128], ones
                )
        return carry

    out_cps = []
    for hf in range(2):
        for tr in range(ETILES):
            lax.fori_loop(
                tr * 8 + 4 * hf, tr * 8 + 4 * hf + 4, zero_body, 0, unroll=2
            )
        for cp in gathers[hf]:
            cp.wait()
        lax.fori_loop(
            hf * (BPW // 2 // LANES),
            (hf + 1) * (BPW // 2 // LANES),
            scat_body,
            0,
            unroll=4,
        )
        for tr in range(ETILES):
            out_cps.append(
                pltpu.async_copy(
                    out_v.at[pl.ds(tr * 8 + 4 * hf, 4)],
                    out_hbm.at[tr, pl.ds(wid * 8 + 4 * hf, 4)],
                    osem,
                )
            )
    for cp in out_cps:
        cp.wait()


@jax.jit
def _run(input, hash_table):
    ids = input.reshape(NW, NCHUNK, IDX_CHUNK)
    t8 = hash_table.astype(jnp.int8)
    w0 = lax.bitcast_convert_type(t8[:, 0:4], jnp.int32)
    w1 = lax.bitcast_convert_type(t8[:, 4:8], jnp.int32)
    mesh = plsc.VectorSubcoreMesh(
        core_axis_name="c",
        subcore_axis_name="s",
        num_cores=NUM_CORES,
        num_subcores=NUM_SUBCORES,
    )
    out = pl.kernel(
        _body,
        out_type=jax.ShapeDtypeStruct((ETILES, BS // 128, 8, 128), jnp.int32),
        mesh=mesh,
        compiler_params=pltpu.CompilerParams(
            use_tc_tiling_on_sc=False, needs_layout_passes=False
        ),
        scratch_types=[
            pltpu.VMEM((NCHUNK, IDX_CHUNK), jnp.int32),
            pltpu.VMEM((BPW,), jnp.int32),
            pltpu.VMEM((BPW,), jnp.int32),
            pltpu.VMEM((NUM_EXPERTS, 8, 128), jnp.int32),
            pltpu.SemaphoreType.DMA,
            pltpu.SemaphoreType.DMA,
            pltpu.SemaphoreType.DMA,
        ],
    )(ids, w0, w1)
    return jnp.transpose(out, (1, 3, 0, 2)).reshape(BS, NUM_EXPERTS)


def kernel(input, hash_table):
    return _run(input, hash_table)


# speedup vs baseline: 1.1231x; 1.1231x over previous
import jax
import jax.numpy as jnp
from jax import lax
from jax.experimental import pallas as pl
from jax.experimental.pallas import tpu as pltpu
from jax.experimental.pallas import tpu_sc as plsc

NUM_EXPERTS = 64
K = 8
BS = 32768
NUM_CORES = 2
NUM_SUBCORES = 16
NW = NUM_CORES * NUM_SUBCORES
BPW = BS // NW
IDX_CHUNK = 128
NCHUNK = BPW // IDX_CHUNK
HCHUNK = NCHUNK // 2
LANES = 16
ETILES = NUM_EXPERTS // 8


def _body(ids_hbm, w0_hbm, w1_hbm, out_hbm, ids_v, b0_v, b1_v, out_v,
          sem_a, sem_b, osem):
    c = lax.axis_index("c")
    s = lax.axis_index("s")
    wid = c * NUM_SUBCORES + s

    pltpu.sync_copy(ids_hbm.at[wid], ids_v)

    gathers = {0: [], 1: []}
    for j in range(NCHUNK):
        hf = j // HCHUNK
        sem = sem_a if hf == 0 else sem_b
        sl = pl.ds(j * IDX_CHUNK, IDX_CHUNK)
        gathers[hf].append(
            pltpu.async_copy(w0_hbm.at[ids_v.at[j]], b0_v.at[sl], sem)
        )
        gathers[hf].append(
            pltpu.async_copy(w1_hbm.at[ids_v.at[j]], b1_v.at[sl], sem)
        )

    zeros = jnp.zeros((LANES,), jnp.int32)

    def zero_body(r, carry):
        for cc in range(8):
            for k in range(8):
                out_v[r, cc, pl.ds(k * LANES, LANES)] = zeros
        return carry

    lax.fori_loop(0, NUM_EXPERTS, zero_body, 0, unroll=2)

    lane = lax.broadcasted_iota(jnp.int32, (LANES,), 0)
    ones = jnp.full((LANES,), 1, jnp.int32)
    m56 = jnp.full((LANES,), 56, jnp.int32)
    m7 = jnp.full((LANES,), 7, jnp.int32)

    def scat_body(i, carry):
        tc = lax.shift_right_logical(i, 3)
        t128 = lane + lax.shift_left(i & 7, 4)
        sl = pl.ds(LANES * i, LANES)
        for bv in (b0_v, b1_v):
            v = bv[sl]
            for m in range(4):
                vs = lax.shift_right_logical(v, 8 * m) if m else v
                plsc.store_scatter(
                    out_v, [(vs & m56) + tc, vs & m7, t128], ones
                )
        return carry

    out_cps = []
    for hf in range(2):
        for cp in gathers[hf]:
            cp.wait()
        lax.fori_loop(
            hf * (BPW // 2 // LANES),
            (hf + 1) * (BPW // 2 // LANES),
            scat_body,
            0,
            unroll=4,
        )
        for tr in range(ETILES):
            out_cps.append(
                pltpu.async_copy(
                    out_v.at[pl.ds(tr * 8 + 4 * hf, 4)],
                    out_hbm.at[tr, pl.ds(wid * 8 + 4 * hf, 4)],
                    osem,
                )
            )
    for cp in out_cps:
        cp.wait()


@jax.jit
def _run(input, hash_table):
    ids = input.reshape(NW, NCHUNK, IDX_CHUNK)
    ht = hash_table
    w0 = ht[:, 0] | (ht[:, 1] << 8) | (ht[:, 2] << 16) | (ht[:, 3] << 24)
    w1 = ht[:, 4] | (ht[:, 5] << 8) | (ht[:, 6] << 16) | (ht[:, 7] << 24)
    mesh = plsc.VectorSubcoreMesh(
        core_axis_name="c",
        subcore_axis_name="s",
        num_cores=NUM_CORES,
        num_subcores=NUM_SUBCORES,
    )
    out = pl.kernel(
        _body,
        out_type=jax.ShapeDtypeStruct((ETILES, BS // 128, 8, 128), jnp.int32),
        mesh=mesh,
        compiler_params=pltpu.CompilerParams(
            use_tc_tiling_on_sc=False, needs_layout_passes=False
        ),
        scratch_types=[
            pltpu.VMEM((NCHUNK, IDX_CHUNK), jnp.int32),
            pltpu.VMEM((BPW,), jnp.int32),
            pltpu.VMEM((BPW,), jnp.int32),
            pltpu.VMEM((NUM_EXPERTS, 8, 128), jnp.int32),
            pltpu.SemaphoreType.DMA,
            pltpu.SemaphoreType.DMA,
            pltpu.SemaphoreType.DMA,
        ],
    )(ids, w0, w1)
    return jnp.transpose(out, (1, 3, 0, 2)).reshape(BS, NUM_EXPERTS)


def kernel(input, hash_table):
    return _run(input, hash_table)


# speedup vs baseline: 1.3212x; 1.1764x over previous
import jax
import jax.numpy as jnp
from jax import lax
from jax.experimental import pallas as pl
from jax.experimental.pallas import tpu as pltpu
from jax.experimental.pallas import tpu_sc as plsc

NUM_EXPERTS = 64
K = 8
BS = 32768
NUM_CORES = 2
NUM_SUBCORES = 16
NW = NUM_CORES * NUM_SUBCORES
BPW = BS // NW
IDX_CHUNK = 128
NCHUNK = BPW // IDX_CHUNK
HCHUNK = NCHUNK // 2
LANES = 16
ETILES = NUM_EXPERTS // 8


def _body(ids_hbm, w0_hbm, w1_hbm, out_hbm, ids_v, b0_v, b1_v, out_v,
          sem_a, sem_b, osem):
    c = lax.axis_index("c")
    s = lax.axis_index("s")
    wid = c * NUM_SUBCORES + s

    pltpu.sync_copy(ids_hbm.at[wid], ids_v)

    gathers = {0: [], 1: []}
    for j in range(NCHUNK):
        hf = j // HCHUNK
        sem = sem_a if hf == 0 else sem_b
        sl = pl.ds(j * IDX_CHUNK, IDX_CHUNK)
        gathers[hf].append(
            pltpu.async_copy(w0_hbm.at[ids_v.at[j]], b0_v.at[sl], sem)
        )
        gathers[hf].append(
            pltpu.async_copy(w1_hbm.at[ids_v.at[j]], b1_v.at[sl], sem)
        )

    zeros = jnp.zeros((LANES,), jnp.int32)

    def zero_body(r, carry):
        for cc in range(8):
            for k in range(8):
                out_v[r, cc, pl.ds(k * LANES, LANES)] = zeros
        return carry

    lax.fori_loop(0, NUM_EXPERTS, zero_body, 0, unroll=2)

    lane = lax.broadcasted_iota(jnp.int32, (LANES,), 0)
    ones = jnp.full((LANES,), 1, jnp.int32)
    m56 = jnp.full((LANES,), 56, jnp.int32)
    m7 = jnp.full((LANES,), 7, jnp.int32)

    def scat_body(i, carry):
        tc = lax.shift_right_logical(i, 3)
        t128 = lane + lax.shift_left(i & 7, 4)
        sl = pl.ds(LANES * i, LANES)
        for bv in (b0_v, b1_v):
            v = bv[sl]
            for m in range(4):
                vs = lax.shift_right_logical(v, 8 * m) if m else v
                plsc.store_scatter(
                    out_v, [(vs & m56) + tc, vs & m7, t128], ones
                )
        return carry

    out_cps = []
    for hf in range(2):
        for cp in gathers[hf]:
            cp.wait()
        lax.fori_loop(
            hf * (BPW // 2 // LANES),
            (hf + 1) * (BPW // 2 // LANES),
            scat_body,
            0,
            unroll=4,
        )
        for tr in range(ETILES):
            out_cps.append(
                pltpu.async_copy(
                    out_v.at[pl.ds(tr * 8 + 4 * hf, 4)],
                    out_hbm.at[tr, pl.ds(wid * 8 + 4 * hf, 4)],
                    osem,
                )
            )
    for cp in out_cps:
        cp.wait()


@jax.jit
def _run(input, hash_table):
    ids = input.reshape(NW, NCHUNK, IDX_CHUNK)
    t8 = hash_table.astype(jnp.int8)
    w0 = lax.bitcast_convert_type(t8[:, 0:4], jnp.int32)
    w1 = lax.bitcast_convert_type(t8[:, 4:8], jnp.int32)
    mesh = plsc.VectorSubcoreMesh(
        core_axis_name="c",
        subcore_axis_name="s",
        num_cores=NUM_CORES,
        num_subcores=NUM_SUBCORES,
    )
    out = pl.kernel(
        _body,
        out_type=jax.ShapeDtypeStruct((ETILES, BS // 128, 8, 128), jnp.int32),
        mesh=mesh,
        compiler_params=pltpu.CompilerParams(
            use_tc_tiling_on_sc=False, needs_layout_passes=False
        ),
        scratch_types=[
            pltpu.VMEM((NCHUNK, IDX_CHUNK), jnp.int32),
            pltpu.VMEM((BPW,), jnp.int32),
            pltpu.VMEM((BPW,), jnp.int32),
            pltpu.VMEM((NUM_EXPERTS, 8, 128), jnp.int32),
            pltpu.SemaphoreType.DMA,
            pltpu.SemaphoreType.DMA,
            pltpu.SemaphoreType.DMA,
        ],
    )(ids, w0, w1)
    return jnp.transpose(out, (1, 3, 0, 2)).reshape(BS, NUM_EXPERTS)


def kernel(input, hash_table):
    return _run(input, hash_table)


# speedup vs baseline: 1.3396x; 1.0139x over previous
import jax
import jax.numpy as jnp
from jax import lax
from jax.experimental import pallas as pl
from jax.experimental.pallas import tpu as pltpu
from jax.experimental.pallas import tpu_sc as plsc

NUM_EXPERTS = 64
K = 8
BS = 32768
NUM_CORES = 2
NUM_SUBCORES = 16
NW = NUM_CORES * NUM_SUBCORES
BPW = BS // NW
IDX_CHUNK = 128
NCHUNK = BPW // IDX_CHUNK
HCHUNK = NCHUNK // 2
LANES = 16
ETILES = NUM_EXPERTS // 8


def _body(ids_hbm, w0_hbm, w1_hbm, out_hbm, ids_v, b0_v, b1_v, out_v,
          sem_a, sem_b, osem):
    c = lax.axis_index("c")
    s = lax.axis_index("s")
    wid = c * NUM_SUBCORES + s

    pltpu.sync_copy(ids_hbm.at[wid], ids_v)

    gathers = {0: [], 1: []}
    for j in range(NCHUNK):
        hf = j // HCHUNK
        sem = sem_a if hf == 0 else sem_b
        sl = pl.ds(j * IDX_CHUNK, IDX_CHUNK)
        gathers[hf].append(
            pltpu.async_copy(w0_hbm.at[ids_v.at[j]], b0_v.at[sl], sem)
        )
        gathers[hf].append(
            pltpu.async_copy(w1_hbm.at[ids_v.at[j]], b1_v.at[sl], sem)
        )

    zeros = jnp.zeros((LANES,), jnp.int32)

    def zero_body(r, carry):
        for cc in range(8):
            for k in range(8):
                out_v[r, cc, pl.ds(k * LANES, LANES)] = zeros
        return carry

    lax.fori_loop(0, NUM_EXPERTS, zero_body, 0, unroll=2)

    lane = lax.broadcasted_iota(jnp.int32, (LANES,), 0)
    ones = jnp.full((LANES,), 1, jnp.int32)
    m56 = jnp.full((LANES,), 56, jnp.int32)
    m7 = jnp.full((LANES,), 7, jnp.int32)

    def scat_body(i, carry):
        tc = lax.shift_right_logical(i, 3)
        t128 = lane + lax.shift_left(i & 7, 4)
        sl = pl.ds(LANES * i, LANES)
        for bv in (b0_v, b1_v):
            v = bv[sl]
            for m in range(4):
                vs = lax.shift_right_logical(v, 8 * m) if m else v
                plsc.store_scatter(
                    out_v, [(vs & m56) + tc, vs & m7, t128], ones
                )
        return carry

    out_cps = []
    for hf in range(2):
        for cp in gathers[hf]:
            cp.wait()
        lax.fori_loop(
            hf * (BPW // 2 // LANES),
            (hf + 1) * (BPW // 2 // LANES),
            scat_body,
            0,
            unroll=4,
        )
        for tr in range(ETILES):
            out_cps.append(
                pltpu.async_copy(
                    out_v.at[pl.ds(tr * 8 + 4 * hf, 4)],
                    out_hbm.at[tr, pl.ds(wid * 8 + 4 * hf, 4)],
                    osem,
                )
            )
    for cp in out_cps:
        cp.wait()


@jax.jit
def _run(input, hash_table):
    ids = input.reshape(NW, NCHUNK, IDX_CHUNK)
    shifts = jnp.array([0, 8, 16, 24], jnp.int32)
    w0 = (hash_table[:, 0:4] << shifts).sum(axis=1)
    w1 = (hash_table[:, 4:8] << shifts).sum(axis=1)
    mesh = plsc.VectorSubcoreMesh(
        core_axis_name="c",
        subcore_axis_name="s",
        num_cores=NUM_CORES,
        num_subcores=NUM_SUBCORES,
    )
    out = pl.kernel(
        _body,
        out_type=jax.ShapeDtypeStruct((ETILES, BS // 128, 8, 128), jnp.int32),
        mesh=mesh,
        compiler_params=pltpu.CompilerParams(
            use_tc_tiling_on_sc=False, needs_layout_passes=False
        ),
        scratch_types=[
            pltpu.VMEM((NCHUNK, IDX_CHUNK), jnp.int32),
            pltpu.VMEM((BPW,), jnp.int32),
            pltpu.VMEM((BPW,), jnp.int32),
            pltpu.VMEM((NUM_EXPERTS, 8, 128), jnp.int32),
            pltpu.SemaphoreType.DMA,
            pltpu.SemaphoreType.DMA,
            pltpu.SemaphoreType.DMA,
        ],
    )(ids, w0, w1)
    return jnp.transpose(out, (1, 3, 0, 2)).reshape(BS, NUM_EXPERTS)


def kernel(input, hash_table):
    return _run(input, hash_table)
